# baseline (speedup 1.0000x reference)
import jax
import jax.numpy as jnp
from jax.experimental import pallas as pl

_ALPHA = 0.1


def _pick_blk(n, want):
    if n % want == 0:
        return want
    return n


def _mm_body(x_ref, w_ref, o_ref):
    o_ref[...] = jnp.dot(
        x_ref[...], w_ref[...], preferred_element_type=jnp.float32
    ).astype(o_ref.dtype)


def _mm(x, w):
    n, d = x.shape
    h = w.shape[1]
    blk = _pick_blk(n, 1000)
    return pl.pallas_call(
        _mm_body,
        grid=(n // blk,),
        in_specs=[
            pl.BlockSpec((blk, d), lambda i: (i, 0)),
            pl.BlockSpec((d, h), lambda i: (0, 0)),
        ],
        out_specs=pl.BlockSpec((blk, h), lambda i: (i, 0)),
        out_shape=jax.ShapeDtypeStruct((n, h), jnp.float32),
    )(x, w)


def _spmm_body(adj_ref, y_ref, b_ref, o_ref):
    acc = jnp.dot(adj_ref[...], y_ref[...], preferred_element_type=jnp.float32)
    acc = acc + b_ref[...]
    o_ref[...] = jnp.where(acc >= 0.0, acc, _ALPHA * acc)


def _spmm(adj, y, b):
    m, k = adj.shape
    h = y.shape[1]
    blk = _pick_blk(m, 80)
    return pl.pallas_call(
        _spmm_body,
        grid=(m // blk,),
        in_specs=[
            pl.BlockSpec((blk, k), lambda i: (i, 0)),
            pl.BlockSpec((k, h), lambda i: (0, 0)),
            pl.BlockSpec((1, h), lambda i: (0, 0)),
        ],
        out_specs=pl.BlockSpec((blk, h), lambda i: (i, 0)),
        out_shape=jax.ShapeDtypeStruct((m, h), jnp.float32),
    )(adj, y, b.reshape(1, h))


def _union_relu_body(a_ref, c_ref, w1_ref, w2_ref, b_ref, o_ref):
    acc = jnp.dot(a_ref[...], w1_ref[...], preferred_element_type=jnp.float32)
    acc += jnp.dot(c_ref[...], w2_ref[...], preferred_element_type=jnp.float32)
    acc += b_ref[...]
    o_ref[...] = jnp.maximum(acc, 0.0)


def _union_lin_body(a_ref, c_ref, w1_ref, w2_ref, b_ref, o_ref):
    acc = jnp.dot(a_ref[...], w1_ref[...], preferred_element_type=jnp.float32)
    acc += jnp.dot(c_ref[...], w2_ref[...], preferred_element_type=jnp.float32)
    acc += b_ref[...]
    o_ref[...] = acc


def _union(a, c, w, b, relu):
    n, da = a.shape
    dc = c.shape[1]
    h = w.shape[1]
    w1 = w[:da]
    w2 = w[da:]
    blk = _pick_blk(n, 1000)
    body = _union_relu_body if relu else _union_lin_body
    return pl.pallas_call(
        body,
        grid=(n // blk,),
        in_specs=[
            pl.BlockSpec((blk, da), lambda i: (i, 0)),
            pl.BlockSpec((blk, dc), lambda i: (i, 0)),
            pl.BlockSpec((da, h), lambda i: (0, 0)),
            pl.BlockSpec((dc, h), lambda i: (0, 0)),
            pl.BlockSpec((1, h), lambda i: (0, 0)),
        ],
        out_specs=pl.BlockSpec((blk, h), lambda i: (i, 0)),
        out_shape=jax.ShapeDtypeStruct((n, h), jnp.float32),
    )(a, c, w1, w2, b.reshape(1, h))


def _gcn(x, adj, w, b):
    return _spmm(adj, _mm(x, w), b)


def kernel(ufea, vfea, UV_adj, VU_adj, d_gc1_w, d_gc1_b, d_gc2_w, d_gc2_b, d_gc3_w, d_gc3_b, d_gc4_w, d_gc4_b, l_gc1_w, l_gc1_b, l_gc2_w, l_gc2_b, l_gc3m_w, l_gc3m_b, l_gc3s_w, l_gc3s_b, l_gc4m_w, l_gc4m_b, l_gc4s_w, l_gc4s_b, d_uu_w, d_uu_b, d_iu_w, d_iu_b, l_uum_w, l_uum_b, l_uus_w, l_uus_b, l_ium_w, l_ium_b, l_ius_w, l_ius_b):
    uho = _gcn(ufea, VU_adj, d_gc1_w, d_gc1_b)
    iho = _gcn(vfea, UV_adj, d_gc2_w, d_gc2_b)
    uho = _gcn(uho, UV_adj, d_gc3_w, d_gc3_b)
    iho = _gcn(iho, VU_adj, d_gc4_w, d_gc4_b)
    u = _union(uho, ufea, d_uu_w, d_uu_b, relu=True)
    v = _union(iho, vfea, d_iu_w, d_iu_b, relu=True)
    uho = _gcn(u, VU_adj, l_gc1_w, l_gc1_b)
    u_mean = _gcn(uho, UV_adj, l_gc3m_w, l_gc3m_b)
    user = _union(u_mean, u, l_uum_w, l_uum_b, relu=False)
    iho = _gcn(v, UV_adj, l_gc2_w, l_gc2_b)
    i_mean = _gcn(iho, VU_adj, l_gc4m_w, l_gc4m_b)
    item = _union(i_mean, v, l_ium_w, l_ium_b, relu=False)
    return user, item


# speedup vs baseline: 1.3233x; 1.3233x over previous
import jax
import jax.numpy as jnp
from jax.experimental import pallas as pl

_ALPHA = 0.1


def _pick_blk(n, want):
    if n % want == 0:
        return want
    return n


def _mm_body(x_ref, w_ref, o_ref):
    o_ref[...] = jnp.dot(
        x_ref[...], w_ref[...], preferred_element_type=jnp.float32
    ).astype(o_ref.dtype)


def _mm(x, w):
    n, d = x.shape
    h = w.shape[1]
    blk = _pick_blk(n, 1000)
    return pl.pallas_call(
        _mm_body,
        grid=(n // blk,),
        in_specs=[
            pl.BlockSpec((blk, d), lambda i: (i, 0)),
            pl.BlockSpec((d, h), lambda i: (0, 0)),
        ],
        out_specs=pl.BlockSpec((blk, h), lambda i: (i, 0)),
        out_shape=jax.ShapeDtypeStruct((n, h), jnp.float32),
    )(x, w)


def _spmm_body(adj_ref, y_ref, b_ref, o_ref):
    acc = jnp.dot(adj_ref[...], y_ref[...], preferred_element_type=jnp.float32)
    acc = acc + b_ref[...]
    o_ref[...] = jnp.where(acc >= 0.0, acc, _ALPHA * acc)


def _spmm(adj, y, b):
    m, k = adj.shape
    h = y.shape[1]
    blk = _pick_blk(m, 400)
    return pl.pallas_call(
        _spmm_body,
        grid=(m // blk,),
        in_specs=[
            pl.BlockSpec((blk, k), lambda i: (i, 0)),
            pl.BlockSpec((k, h), lambda i: (0, 0)),
            pl.BlockSpec((1, h), lambda i: (0, 0)),
        ],
        out_specs=pl.BlockSpec((blk, h), lambda i: (i, 0)),
        out_shape=jax.ShapeDtypeStruct((m, h), jnp.float32),
    )(adj, y, b.reshape(1, h))


def _union_relu_body(a_ref, c_ref, w1_ref, w2_ref, b_ref, o_ref):
    acc = jnp.dot(a_ref[...], w1_ref[...], preferred_element_type=jnp.float32)
    acc += jnp.dot(c_ref[...], w2_ref[...], preferred_element_type=jnp.float32)
    acc += b_ref[...]
    o_ref[...] = jnp.maximum(acc, 0.0)


def _union_lin_body(a_ref, c_ref, w1_ref, w2_ref, b_ref, o_ref):
    acc = jnp.dot(a_ref[...], w1_ref[...], preferred_element_type=jnp.float32)
    acc += jnp.dot(c_ref[...], w2_ref[...], preferred_element_type=jnp.float32)
    acc += b_ref[...]
    o_ref[...] = acc


def _union(a, c, w, b, relu):
    n, da = a.shape
    dc = c.shape[1]
    h = w.shape[1]
    w1 = w[:da]
    w2 = w[da:]
    blk = _pick_blk(n, 1000)
    body = _union_relu_body if relu else _union_lin_body
    return pl.pallas_call(
        body,
        grid=(n // blk,),
        in_specs=[
            pl.BlockSpec((blk, da), lambda i: (i, 0)),
            pl.BlockSpec((blk, dc), lambda i: (i, 0)),
            pl.BlockSpec((da, h), lambda i: (0, 0)),
            pl.BlockSpec((dc, h), lambda i: (0, 0)),
            pl.BlockSpec((1, h), lambda i: (0, 0)),
        ],
        out_specs=pl.BlockSpec((blk, h), lambda i: (i, 0)),
        out_shape=jax.ShapeDtypeStruct((n, h), jnp.float32),
    )(a, c, w1, w2, b.reshape(1, h))


def _gcn(x, adj, w, b):
    return _spmm(adj, _mm(x, w), b)


def kernel(ufea, vfea, UV_adj, VU_adj, d_gc1_w, d_gc1_b, d_gc2_w, d_gc2_b, d_gc3_w, d_gc3_b, d_gc4_w, d_gc4_b, l_gc1_w, l_gc1_b, l_gc2_w, l_gc2_b, l_gc3m_w, l_gc3m_b, l_gc3s_w, l_gc3s_b, l_gc4m_w, l_gc4m_b, l_gc4s_w, l_gc4s_b, d_uu_w, d_uu_b, d_iu_w, d_iu_b, l_uum_w, l_uum_b, l_uus_w, l_uus_b, l_ium_w, l_ium_b, l_ius_w, l_ius_b):
    uho = _gcn(ufea, VU_adj, d_gc1_w, d_gc1_b)
    iho = _gcn(vfea, UV_adj, d_gc2_w, d_gc2_b)
    uho = _gcn(uho, UV_adj, d_gc3_w, d_gc3_b)
    iho = _gcn(iho, VU_adj, d_gc4_w, d_gc4_b)
    u = _union(uho, ufea, d_uu_w, d_uu_b, relu=True)
    v = _union(iho, vfea, d_iu_w, d_iu_b, relu=True)
    uho = _gcn(u, VU_adj, l_gc1_w, l_gc1_b)
    u_mean = _gcn(uho, UV_adj, l_gc3m_w, l_gc3m_b)
    user = _union(u_mean, u, l_uum_w, l_uum_b, relu=False)
    iho = _gcn(v, UV_adj, l_gc2_w, l_gc2_b)
    i_mean = _gcn(iho, VU_adj, l_gc4m_w, l_gc4m_b)
    item = _union(i_mean, v, l_ium_w, l_ium_b, relu=False)
    return user, item


# speedup vs baseline: 1.6151x; 1.2205x over previous
import jax
import jax.numpy as jnp
from jax.experimental import pallas as pl

_ALPHA = 0.1


def _pick_blk(n, want):
    if n % want == 0:
        return want
    return n


def _mm_body(x_ref, w_ref, o_ref):
    o_ref[...] = jnp.dot(
        x_ref[...], w_ref[...], preferred_element_type=jnp.float32
    ).astype(o_ref.dtype)


def _mm(x, w, out_dtype=jnp.float32):
    n, d = x.shape
    h = w.shape[1]
    blk = _pick_blk(n, 1000)
    return pl.pallas_call(
        _mm_body,
        grid=(n // blk,),
        in_specs=[
            pl.BlockSpec((blk, d), lambda i: (i, 0)),
            pl.BlockSpec((d, h), lambda i: (0, 0)),
        ],
        out_specs=pl.BlockSpec((blk, h), lambda i: (i, 0)),
        out_shape=jax.ShapeDtypeStruct((n, h), out_dtype),
    )(x, w)


def _spmm_body(adj_ref, y_ref, b_ref, o_ref):
    acc = jnp.dot(adj_ref[...], y_ref[...], preferred_element_type=jnp.float32)
    acc = acc + b_ref[...]
    o_ref[...] = jnp.where(acc >= 0.0, acc, _ALPHA * acc)


def _spmm(adj, y, b, blk_rows=400):
    m, k = adj.shape
    h = y.shape[1]
    blk = _pick_blk(m, blk_rows)
    return pl.pallas_call(
        _spmm_body,
        grid=(m // blk,),
        in_specs=[
            pl.BlockSpec((blk, k), lambda i: (i, 0)),
            pl.BlockSpec((k, h), lambda i: (0, 0)),
            pl.BlockSpec((1, h), lambda i: (0, 0)),
        ],
        out_specs=pl.BlockSpec((blk, h), lambda i: (i, 0)),
        out_shape=jax.ShapeDtypeStruct((m, h), jnp.float32),
    )(adj, y, b.reshape(1, h))


def _spmm_cache_body(adj_ref, y_ref, b_ref, o_ref, adjb_ref):
    a = adj_ref[...]
    acc = jnp.dot(a, y_ref[...], preferred_element_type=jnp.float32)
    acc = acc + b_ref[...]
    o_ref[...] = jnp.where(acc >= 0.0, acc, _ALPHA * acc)
    adjb_ref[...] = a.astype(jnp.bfloat16)


def _spmm_cache(adj, y, b, blk_rows=200):
    m, k = adj.shape
    h = y.shape[1]
    blk = _pick_blk(m, blk_rows)
    return pl.pallas_call(
        _spmm_cache_body,
        grid=(m // blk,),
        in_specs=[
            pl.BlockSpec((blk, k), lambda i: (i, 0)),
            pl.BlockSpec((k, h), lambda i: (0, 0)),
            pl.BlockSpec((1, h), lambda i: (0, 0)),
        ],
        out_specs=[
            pl.BlockSpec((blk, h), lambda i: (i, 0)),
            pl.BlockSpec((blk, k), lambda i: (i, 0)),
        ],
        out_shape=[
            jax.ShapeDtypeStruct((m, h), jnp.float32),
            jax.ShapeDtypeStruct((m, k), jnp.bfloat16),
        ],
    )(adj, y, b.reshape(1, h))


def _union_relu_body(a_ref, c_ref, w1_ref, w2_ref, b_ref, o_ref):
    acc = jnp.dot(a_ref[...], w1_ref[...], preferred_element_type=jnp.float32)
    acc += jnp.dot(c_ref[...], w2_ref[...], preferred_element_type=jnp.float32)
    acc += b_ref[...]
    o_ref[...] = jnp.maximum(acc, 0.0)


def _union_lin_body(a_ref, c_ref, w1_ref, w2_ref, b_ref, o_ref):
    acc = jnp.dot(a_ref[...], w1_ref[...], preferred_element_type=jnp.float32)
    acc += jnp.dot(c_ref[...], w2_ref[...], preferred_element_type=jnp.float32)
    acc += b_ref[...]
    o_ref[...] = acc


def _union(a, c, w, b, relu):
    n, da = a.shape
    dc = c.shape[1]
    h = w.shape[1]
    w1 = w[:da]
    w2 = w[da:]
    blk = _pick_blk(n, 1000)
    body = _union_relu_body if relu else _union_lin_body
    return pl.pallas_call(
        body,
        grid=(n // blk,),
        in_specs=[
            pl.BlockSpec((blk, da), lambda i: (i, 0)),
            pl.BlockSpec((blk, dc), lambda i: (i, 0)),
            pl.BlockSpec((da, h), lambda i: (0, 0)),
            pl.BlockSpec((dc, h), lambda i: (0, 0)),
            pl.BlockSpec((1, h), lambda i: (0, 0)),
        ],
        out_specs=pl.BlockSpec((blk, h), lambda i: (i, 0)),
        out_shape=jax.ShapeDtypeStruct((n, h), jnp.float32),
    )(a, c, w1, w2, b.reshape(1, h))


def _gcn_bf16(x, adj_bf16, w, b):
    return _spmm(adj_bf16, _mm(x, w, out_dtype=jnp.bfloat16), b, blk_rows=1000)


def kernel(ufea, vfea, UV_adj, VU_adj, d_gc1_w, d_gc1_b, d_gc2_w, d_gc2_b, d_gc3_w, d_gc3_b, d_gc4_w, d_gc4_b, l_gc1_w, l_gc1_b, l_gc2_w, l_gc2_b, l_gc3m_w, l_gc3m_b, l_gc3s_w, l_gc3s_b, l_gc4m_w, l_gc4m_b, l_gc4s_w, l_gc4s_b, d_uu_w, d_uu_b, d_iu_w, d_iu_b, l_uum_w, l_uum_b, l_uus_w, l_uus_b, l_ium_w, l_ium_b, l_ius_w, l_ius_b):
    uho, VU_bf = _spmm_cache(VU_adj, _mm(ufea, d_gc1_w), d_gc1_b)
    iho, UV_bf = _spmm_cache(UV_adj, _mm(vfea, d_gc2_w), d_gc2_b)
    uho = _gcn_bf16(uho, UV_bf, d_gc3_w, d_gc3_b)
    iho = _gcn_bf16(iho, VU_bf, d_gc4_w, d_gc4_b)
    u = _union(uho, ufea, d_uu_w, d_uu_b, relu=True)
    v = _union(iho, vfea, d_iu_w, d_iu_b, relu=True)
    uho = _gcn_bf16(u, VU_bf, l_gc1_w, l_gc1_b)
    u_mean = _gcn_bf16(uho, UV_bf, l_gc3m_w, l_gc3m_b)
    user = _union(u_mean, u, l_uum_w, l_uum_b, relu=False)
    iho = _gcn_bf16(v, UV_bf, l_gc2_w, l_gc2_b)
    i_mean = _gcn_bf16(iho, VU_bf, l_gc4m_w, l_gc4m_b)
    item = _union(i_mean, v, l_ium_w, l_ium_b, relu=False)
    return user, item


# speedup vs baseline: 1.7814x; 1.1030x over previous
import jax
import jax.numpy as jnp
from jax.experimental import pallas as pl

_ALPHA = 0.1
_BF = jnp.bfloat16


def _pick_blk(n, want):
    return want if n % want == 0 else n


def _dot(a, b):
    return jnp.dot(a, b, preferred_element_type=jnp.float32)


def _lrelu(x):
    return jnp.where(x >= 0.0, x, _ALPHA * x)


def _mm_body(x_ref, w_ref, o_ref):
    o_ref[...] = _dot(x_ref[...], w_ref[...]).astype(o_ref.dtype)


def _mm(x, w):
    n, d = x.shape
    h = w.shape[1]
    blk = _pick_blk(n, 1000)
    return pl.pallas_call(
        _mm_body,
        grid=(n // blk,),
        in_specs=[
            pl.BlockSpec((blk, d), lambda i: (i, 0)),
            pl.BlockSpec((d, h), lambda i: (0, 0)),
        ],
        out_specs=pl.BlockSpec((blk, h), lambda i: (i, 0)),
        out_shape=jax.ShapeDtypeStruct((n, h), jnp.float32),
    )(x, w)


def _spmm_a_body(adj_ref, y_ref, b_ref, wn_ref, yn_ref, adjb_ref):
    a = adj_ref[...]
    h = _lrelu(_dot(a, y_ref[...]) + b_ref[...])
    yn_ref[...] = _dot(h, wn_ref[...]).astype(_BF)
    adjb_ref[...] = a.astype(_BF)


def _spmm_a(adj, y, b, w_next):
    m, k = adj.shape
    h = y.shape[1]
    hn = w_next.shape[1]
    blk = _pick_blk(m, 200)
    return pl.pallas_call(
        _spmm_a_body,
        grid=(m // blk,),
        in_specs=[
            pl.BlockSpec((blk, k), lambda i: (i, 0)),
            pl.BlockSpec((k, h), lambda i: (0, 0)),
            pl.BlockSpec((1, h), lambda i: (0, 0)),
            pl.BlockSpec((h, hn), lambda i: (0, 0)),
        ],
        out_specs=[
            pl.BlockSpec((blk, hn), lambda i: (i, 0)),
            pl.BlockSpec((blk, k), lambda i: (i, 0)),
        ],
        out_shape=[
            jax.ShapeDtypeStruct((m, hn), _BF),
            jax.ShapeDtypeStruct((m, k), _BF),
        ],
    )(adj, y, b.reshape(1, h), w_next)


def _spmm_b_body(adj_ref, y_ref, b_ref, feat_ref, wu1_ref, wu2_ref, bu_ref,
                 wn_ref, u_ref, yn_ref):
    h = _lrelu(_dot(adj_ref[...], y_ref[...]) + b_ref[...])
    u = _dot(h, wu1_ref[...]) + _dot(feat_ref[...], wu2_ref[...]) + bu_ref[...]
    u = jnp.maximum(u, 0.0)
    u_ref[...] = u
    yn_ref[...] = _dot(u, wn_ref[...]).astype(_BF)


def _spmm_b(adj_bf, y, b, feat, wu, bu, w_next):
    m, k = adj_bf.shape
    h = y.shape[1]
    df = feat.shape[1]
    hu = wu.shape[1]
    hn = w_next.shape[1]
    blk = _pick_blk(m, 1000)
    return pl.pallas_call(
        _spmm_b_body,
        grid=(m // blk,),
        in_specs=[
            pl.BlockSpec((blk, k), lambda i: (i, 0)),
            pl.BlockSpec((k, h), lambda i: (0, 0)),
            pl.BlockSpec((1, h), lambda i: (0, 0)),
            pl.BlockSpec((blk, df), lambda i: (i, 0)),
            pl.BlockSpec((h, hu), lambda i: (0, 0)),
            pl.BlockSpec((df, hu), lambda i: (0, 0)),
            pl.BlockSpec((1, hu), lambda i: (0, 0)),
            pl.BlockSpec((hu, hn), lambda i: (0, 0)),
        ],
        out_specs=[
            pl.BlockSpec((blk, hu), lambda i: (i, 0)),
            pl.BlockSpec((blk, hn), lambda i: (i, 0)),
        ],
        out_shape=[
            jax.ShapeDtypeStruct((m, hu), jnp.float32),
            jax.ShapeDtypeStruct((m, hn), _BF),
        ],
    )(adj_bf, y, b.reshape(1, h), feat, wu[:h], wu[h:], bu.reshape(1, hu),
      w_next)


def _spmm_c_body(adj_ref, y_ref, b_ref, wn_ref, yn_ref):
    h = _lrelu(_dot(adj_ref[...], y_ref[...]) + b_ref[...])
    yn_ref[...] = _dot(h, wn_ref[...]).astype(_BF)


def _spmm_c(adj_bf, y, b, w_next):
    m, k = adj_bf.shape
    h = y.shape[1]
    hn = w_next.shape[1]
    blk = _pick_blk(m, 1000)
    return pl.pallas_call(
        _spmm_c_body,
        grid=(m // blk,),
        in_specs=[
            pl.BlockSpec((blk, k), lambda i: (i, 0)),
            pl.BlockSpec((k, h), lambda i: (0, 0)),
            pl.BlockSpec((1, h), lambda i: (0, 0)),
            pl.BlockSpec((h, hn), lambda i: (0, 0)),
        ],
        out_specs=pl.BlockSpec((blk, hn), lambda i: (i, 0)),
        out_shape=jax.ShapeDtypeStruct((m, hn), _BF),
    )(adj_bf, y, b.reshape(1, h), w_next)


def _spmm_d_body(adj_ref, y_ref, b_ref, feat_ref, wu1_ref, wu2_ref, bu_ref,
                 o_ref):
    h = _lrelu(_dot(adj_ref[...], y_ref[...]) + b_ref[...])
    o_ref[...] = (_dot(h, wu1_ref[...]) + _dot(feat_ref[...], wu2_ref[...])
                  + bu_ref[...])


def _spmm_d(adj_bf, y, b, feat, wu, bu):
    m, k = adj_bf.shape
    h = y.shape[1]
    df = feat.shape[1]
    hu = wu.shape[1]
    blk = _pick_blk(m, 1000)
    return pl.pallas_call(
        _spmm_d_body,
        grid=(m // blk,),
        in_specs=[
            pl.BlockSpec((blk, k), lambda i: (i, 0)),
            pl.BlockSpec((k, h), lambda i: (0, 0)),
            pl.BlockSpec((1, h), lambda i: (0, 0)),
            pl.BlockSpec((blk, df), lambda i: (i, 0)),
            pl.BlockSpec((h, hu), lambda i: (0, 0)),
            pl.BlockSpec((df, hu), lambda i: (0, 0)),
            pl.BlockSpec((1, hu), lambda i: (0, 0)),
        ],
        out_specs=pl.BlockSpec((blk, hu), lambda i: (i, 0)),
        out_shape=jax.ShapeDtypeStruct((m, hu), jnp.float32),
    )(adj_bf, y, b.reshape(1, h), feat, wu[:h], wu[h:], bu.reshape(1, hu))


def kernel(ufea, vfea, UV_adj, VU_adj, d_gc1_w, d_gc1_b, d_gc2_w, d_gc2_b, d_gc3_w, d_gc3_b, d_gc4_w, d_gc4_b, l_gc1_w, l_gc1_b, l_gc2_w, l_gc2_b, l_gc3m_w, l_gc3m_b, l_gc3s_w, l_gc3s_b, l_gc4m_w, l_gc4m_b, l_gc4s_w, l_gc4s_b, d_uu_w, d_uu_b, d_iu_w, d_iu_b, l_uum_w, l_uum_b, l_uus_w, l_uus_b, l_ium_w, l_ium_b, l_ius_w, l_ius_b):
    y1 = _mm(ufea, d_gc1_w)
    y2 = _mm(vfea, d_gc2_w)
    y3, VU_bf = _spmm_a(VU_adj, y1, d_gc1_b, d_gc3_w)
    y4, UV_bf = _spmm_a(UV_adj, y2, d_gc2_b, d_gc4_w)
    u, y5 = _spmm_b(UV_bf, y3, d_gc3_b, ufea, d_uu_w, d_uu_b, l_gc1_w)
    v, y6 = _spmm_b(VU_bf, y4, d_gc4_b, vfea, d_iu_w, d_iu_b, l_gc2_w)
    y7 = _spmm_c(VU_bf, y5, l_gc1_b, l_gc3m_w)
    y8 = _spmm_c(UV_bf, y6, l_gc2_b, l_gc4m_w)
    user = _spmm_d(UV_bf, y7, l_gc3m_b, u, l_uum_w, l_uum_b)
    item = _spmm_d(VU_bf, y8, l_gc4m_b, v, l_ium_w, l_ium_b)
    return user, item
